# baseline (speedup 1.0000x reference)
import functools

import jax
import jax.numpy as jnp
from jax import lax
from jax.experimental import pallas as pl
from jax.experimental.pallas import tpu as pltpu
from jax.experimental.pallas import tpu_sc as plsc

_BLK = 16


@functools.lru_cache(maxsize=None)
def _build_gather(P, S2, NSEL, D):
    TOT = P * NSEL
    L = NSEL * _BLK
    NC, NS = 2, 16
    NW = NC * NS
    per_w = TOT // NW
    TPW = per_w * _BLK
    CHT = 128
    n_chunks = TPW // CHT
    cpp = NSEL * _BLK // CHT
    ngrp = per_w // 16
    mesh = plsc.VectorSubcoreMesh(core_axis_name="c", subcore_axis_name="s")

    @functools.partial(
        pl.kernel,
        mesh=mesh,
        out_type=[
            jax.ShapeDtypeStruct((P, L, D), jnp.float32),
            jax.ShapeDtypeStruct((P, L, D), jnp.float32),
        ],
        scratch_types=[
            pltpu.VMEM((per_w,), jnp.int32),
            pltpu.VMEM((TPW,), jnp.int32),
            pltpu.VMEM((CHT, D), jnp.float32),
            pltpu.VMEM((CHT, D), jnp.float32),
            pltpu.VMEM((CHT, D), jnp.float32),
            pltpu.VMEM((CHT, D), jnp.float32),
            pltpu.SemaphoreType.DMA,
        ],
    )
    def gather(sidx, kf_tab, vf_tab, k_out, v_out,
               idxv, tix, kb0, vb0, kb1, vb1, sem):
        wid = lax.axis_index("c") * NS + lax.axis_index("s")
        base = wid * per_w
        pltpu.sync_copy(sidx.at[pl.ds(base, per_w)], idxv)
        for c in range(ngrp):
            sp, c8 = c // (ngrp // 2), c % (ngrp // 2)
            pair_c = wid * 2 + sp
            bids = idxv[pl.ds(c * 16, 16)] * _BLK + pair_c * S2
            for t in range(_BLK):
                tix[pl.ds(sp * (TPW // 2) + t * 128 + c8 * 16, 16)] = bids + t
        kbs, vbs = (kb0, kb1), (vb0, vb1)
        copies = [None, None]
        for g in range(n_chunks + 1):
            if g < n_chunks:
                b = g % 2
                isl = tix.at[pl.ds(g * CHT, CHT)]
                ck = pltpu.async_copy(kf_tab.at[isl], kbs[b], sem)
                cv = pltpu.async_copy(vf_tab.at[isl], vbs[b], sem)
                copies[b] = (ck, cv)
            if g > 0:
                pb = (g - 1) % 2
                ckp, cvp = copies[pb]
                ckp.wait()
                cvp.wait()
                pair = wid * 2 + (g - 1) // cpp
                toff = ((g - 1) % cpp) * CHT
                pltpu.sync_copy(kbs[pb], k_out.at[pair, pl.ds(toff, CHT), :])
                pltpu.sync_copy(vbs[pb], v_out.at[pair, pl.ds(toff, CHT), :])

    return gather


def _attn_body(scale_ref, q_ref, k_ref, v_ref, o_ref):
    q = q_ref[0]
    kf = k_ref[0]
    logits = lax.dot_general(q, kf, (((1,), (1,)), ((), ())),
                             preferred_element_type=jnp.float32)
    logits = logits * scale_ref[0]
    m = jnp.max(logits, axis=-1, keepdims=True)
    e = jnp.exp(logits - m)
    den = jnp.sum(e, axis=-1, keepdims=True)
    o = lax.dot_general(e, v_ref[0], (((1,), (0,)), ((), ())),
                        preferred_element_type=jnp.float32)
    o_ref[0] = o / den


@functools.lru_cache(maxsize=None)
def _build_attn(P, GS, L, D):
    return pl.pallas_call(
        _attn_body,
        grid=(P,),
        in_specs=[
            pl.BlockSpec(memory_space=pltpu.SMEM),
            pl.BlockSpec((1, GS, D), lambda i: (i, 0, 0)),
            pl.BlockSpec((1, L, D), lambda i: (i, 0, 0)),
            pl.BlockSpec((1, L, D), lambda i: (i, 0, 0)),
        ],
        out_specs=pl.BlockSpec((1, GS, D), lambda i: (i, 0, 0)),
        out_shape=jax.ShapeDtypeStruct((P, GS, D), jnp.float32),
    )


def kernel(query, key, value, sparse_indices, key_dequant_scale,
           value_dequant_scale, scale_value, sparse_block_size):
    B, N1, S1, D = query.shape
    _, N2, S2, _ = key.shape
    G = N1 // N2
    NSEL = sparse_indices.shape[-1]
    P = B * N2
    TOT = P * NSEL
    L = NSEL * _BLK
    GS = G * S1

    def pack(x, sc):
        xb = (x.astype(jnp.float32) * sc[..., None]).astype(jnp.bfloat16)
        xb = xb.reshape(P * S2, D)
        lo = lax.bitcast_convert_type(xb[0::2], jnp.uint16).astype(jnp.uint32)
        hi = lax.bitcast_convert_type(xb[1::2], jnp.uint16).astype(jnp.uint32)
        return lax.bitcast_convert_type(lo | (hi << 16), jnp.int32)
    kf_tab = pack(key, key_dequant_scale)
    vf_tab = pack(value, value_dequant_scale)
    sidx = sparse_indices.reshape(TOT)
    return kf_tab, vf_tab

    k_sel, v_sel = _build_gather(P, S2, NSEL, D)(sidx, kf_tab, vf_tab)

    q3 = query.reshape(P, GS, D)
    scale = jnp.asarray(scale_value, jnp.float32).reshape(1)
    out = _build_attn(P, GS, L, D)(scale, q3, k_sel, v_sel)
    return out.reshape(B, N1, S1, D)


# speedup vs baseline: 4.6126x; 4.6126x over previous
import functools

import jax
import jax.numpy as jnp
from jax import lax
from jax.experimental import pallas as pl
from jax.experimental.pallas import tpu as pltpu
from jax.experimental.pallas import tpu_sc as plsc

_BLK = 16


@functools.lru_cache(maxsize=None)
def _build_gather(P, S2, NSEL, D):
    TOT = P * NSEL
    L = NSEL * _BLK
    NC, NS = 2, 16
    NW = NC * NS
    per_w = TOT // NW
    TPW = per_w * _BLK
    CHT = 128
    n_chunks = TPW // CHT
    cpp = NSEL * _BLK // CHT
    ngrp = per_w // 16
    mesh = plsc.VectorSubcoreMesh(core_axis_name="c", subcore_axis_name="s")

    @functools.partial(
        pl.kernel,
        mesh=mesh,
        out_type=[
            jax.ShapeDtypeStruct((P, L, D), jnp.float32),
            jax.ShapeDtypeStruct((P, L, D), jnp.float32),
        ],
        scratch_types=[
            pltpu.VMEM((per_w,), jnp.int32),
            pltpu.VMEM((TPW,), jnp.int32),
            pltpu.VMEM((CHT, D), jnp.float32),
            pltpu.VMEM((CHT, D), jnp.float32),
            pltpu.VMEM((CHT, D), jnp.float32),
            pltpu.VMEM((CHT, D), jnp.float32),
            pltpu.SemaphoreType.DMA,
        ],
    )
    def gather(sidx, kf_tab, vf_tab, k_out, v_out,
               idxv, tix, kb0, vb0, kb1, vb1, sem):
        wid = lax.axis_index("c") * NS + lax.axis_index("s")
        base = wid * per_w
        pltpu.sync_copy(sidx.at[pl.ds(base, per_w)], idxv)
        for c in range(ngrp):
            sp, c8 = c // (ngrp // 2), c % (ngrp // 2)
            pair_c = wid * 2 + sp
            bids = idxv[pl.ds(c * 16, 16)] * _BLK + pair_c * S2
            for t in range(_BLK):
                tix[pl.ds(sp * (TPW // 2) + t * 128 + c8 * 16, 16)] = bids + t
        kbs, vbs = (kb0, kb1), (vb0, vb1)
        copies = [None, None]
        for g in range(n_chunks + 1):
            if g < n_chunks:
                b = g % 2
                isl = tix.at[pl.ds(g * CHT, CHT)]
                ck = pltpu.async_copy(kf_tab.at[isl], kbs[b], sem)
                cv = pltpu.async_copy(vf_tab.at[isl], vbs[b], sem)
                copies[b] = (ck, cv)
            if g > 0:
                pb = (g - 1) % 2
                ckp, cvp = copies[pb]
                ckp.wait()
                cvp.wait()
                pair = wid * 2 + (g - 1) // cpp
                toff = ((g - 1) % cpp) * CHT
                pltpu.sync_copy(kbs[pb], k_out.at[pair, pl.ds(toff, CHT), :])
                pltpu.sync_copy(vbs[pb], v_out.at[pair, pl.ds(toff, CHT), :])

    return gather


def _attn_body(scale_ref, q_ref, k_ref, v_ref, o_ref):
    q = q_ref[0].astype(jnp.bfloat16)
    kf = k_ref[0].astype(jnp.bfloat16)
    logits = lax.dot_general(q, kf, (((1,), (1,)), ((), ())),
                             preferred_element_type=jnp.float32)
    logits = logits * scale_ref[0]
    m = jnp.max(logits, axis=-1, keepdims=True)
    e = jnp.exp(logits - m)
    den = jnp.sum(e, axis=-1, keepdims=True)
    o = lax.dot_general(e.astype(jnp.bfloat16), v_ref[0].astype(jnp.bfloat16),
                        (((1,), (0,)), ((), ())),
                        preferred_element_type=jnp.float32)
    o_ref[0] = o / den


@functools.lru_cache(maxsize=None)
def _build_attn(P, GS, L, D):
    return pl.pallas_call(
        _attn_body,
        grid=(P,),
        in_specs=[
            pl.BlockSpec(memory_space=pltpu.SMEM),
            pl.BlockSpec((1, GS, D), lambda i: (i, 0, 0)),
            pl.BlockSpec((1, L, D), lambda i: (i, 0, 0)),
            pl.BlockSpec((1, L, D), lambda i: (i, 0, 0)),
        ],
        out_specs=pl.BlockSpec((1, GS, D), lambda i: (i, 0, 0)),
        out_shape=jax.ShapeDtypeStruct((P, GS, D), jnp.float32),
    )


def kernel(query, key, value, sparse_indices, key_dequant_scale,
           value_dequant_scale, scale_value, sparse_block_size):
    B, N1, S1, D = query.shape
    _, N2, S2, _ = key.shape
    G = N1 // N2
    NSEL = sparse_indices.shape[-1]
    P = B * N2
    TOT = P * NSEL
    L = NSEL * _BLK
    GS = G * S1

    kf_tab = (key.astype(jnp.float32)
              * key_dequant_scale[..., None]).reshape(P * S2, D)
    vf_tab = (value.astype(jnp.float32)
              * value_dequant_scale[..., None]).reshape(P * S2, D)
    sidx = sparse_indices.reshape(TOT)

    k_sel, v_sel = _build_gather(P, S2, NSEL, D)(sidx, kf_tab, vf_tab)

    q3 = query.reshape(P, GS, D)
    scale = jnp.asarray(scale_value, jnp.float32).reshape(1)
    out = _build_attn(P, GS, L, D)(scale, q3, k_sel, v_sel)
    return out.reshape(B, N1, S1, D)


# speedup vs baseline: 4.6426x; 1.0065x over previous
import functools

import jax
import jax.numpy as jnp
from jax import lax
from jax.experimental import pallas as pl
from jax.experimental.pallas import tpu as pltpu
from jax.experimental.pallas import tpu_sc as plsc

_BLK = 16


@functools.lru_cache(maxsize=None)
def _build_gather(P, S2, NSEL, D):
    TOT = P * NSEL
    L = NSEL * _BLK
    NC, NS = 2, 16
    NW = NC * NS
    assert P == NW
    per_w = NSEL
    TPW = per_w * _BLK
    CHT = 128
    n_chunks = TPW // CHT
    ngrp = per_w // 16
    mesh = plsc.VectorSubcoreMesh(core_axis_name="c", subcore_axis_name="s")

    @functools.partial(
        pl.kernel,
        mesh=mesh,
        out_type=[
            jax.ShapeDtypeStruct((P, L, D), jnp.float32),
            jax.ShapeDtypeStruct((P, L, D), jnp.float32),
        ],
        scratch_types=[
            pltpu.VMEM((per_w,), jnp.int32),
            pltpu.VMEM((TPW,), jnp.int32),
            pltpu.VMEM((CHT, D), jnp.float32),
            pltpu.VMEM((CHT, D), jnp.float32),
            pltpu.VMEM((CHT, D), jnp.float32),
            pltpu.VMEM((CHT, D), jnp.float32),
            pltpu.SemaphoreType.DMA,
        ],
    )
    def gather(sidx, kf_tab, vf_tab, k_out, v_out,
               idxv, tix, kb0, vb0, kb1, vb1, sem):
        wid = lax.axis_index("c") * NS + lax.axis_index("s")
        base = wid * per_w
        pltpu.sync_copy(sidx.at[pl.ds(base, per_w)], idxv)
        for c in range(ngrp):
            bids = idxv[pl.ds(c * 16, 16)] * _BLK + wid * S2
            for t in range(_BLK):
                tix[pl.ds(t * NSEL + c * 16, 16)] = bids + t
        kbs, vbs = (kb0, kb1), (vb0, vb1)
        copies = [None, None]
        for g in range(n_chunks + 1):
            if g < n_chunks:
                b = g % 2
                isl = tix.at[pl.ds(g * CHT, CHT)]
                ck = pltpu.async_copy(kf_tab.at[isl], kbs[b], sem)
                cv = pltpu.async_copy(vf_tab.at[isl], vbs[b], sem)
                copies[b] = (ck, cv)
            if g > 0:
                pb = (g - 1) % 2
                ckp, cvp = copies[pb]
                ckp.wait()
                cvp.wait()
                toff = (g - 1) * CHT
                pltpu.sync_copy(kbs[pb], k_out.at[wid, pl.ds(toff, CHT), :])
                pltpu.sync_copy(vbs[pb], v_out.at[wid, pl.ds(toff, CHT), :])

    return gather


def _attn_body(scale_ref, q_ref, k_ref, v_ref, o_ref):
    q = q_ref[0].astype(jnp.bfloat16)
    kf = k_ref[0].astype(jnp.bfloat16)
    logits = lax.dot_general(q, kf, (((1,), (1,)), ((), ())),
                             preferred_element_type=jnp.float32)
    logits = logits * scale_ref[0]
    m = jnp.max(logits, axis=-1, keepdims=True)
    e = jnp.exp(logits - m)
    den = jnp.sum(e, axis=-1, keepdims=True)
    o = lax.dot_general(e.astype(jnp.bfloat16), v_ref[0].astype(jnp.bfloat16),
                        (((1,), (0,)), ((), ())),
                        preferred_element_type=jnp.float32)
    o_ref[0] = o / den


@functools.lru_cache(maxsize=None)
def _build_attn(P, GS, L, D):
    return pl.pallas_call(
        _attn_body,
        grid=(P,),
        in_specs=[
            pl.BlockSpec(memory_space=pltpu.SMEM),
            pl.BlockSpec((1, GS, D), lambda i: (i, 0, 0)),
            pl.BlockSpec((1, L, D), lambda i: (i, 0, 0)),
            pl.BlockSpec((1, L, D), lambda i: (i, 0, 0)),
        ],
        out_specs=pl.BlockSpec((1, GS, D), lambda i: (i, 0, 0)),
        out_shape=jax.ShapeDtypeStruct((P, GS, D), jnp.float32),
    )


def kernel(query, key, value, sparse_indices, key_dequant_scale,
           value_dequant_scale, scale_value, sparse_block_size):
    B, N1, S1, D = query.shape
    _, N2, S2, _ = key.shape
    G = N1 // N2
    NSEL = sparse_indices.shape[-1]
    P = B * N2
    TOT = P * NSEL
    L = NSEL * _BLK
    GS = G * S1

    scale = jnp.asarray(scale_value, jnp.float32).reshape(1)
    NCH = 2
    BC = B // NCH
    PC = BC * N2
    outs = []
    for c in range(NCH):
        b0 = c * BC
        kf_tab = (key[b0:b0 + BC].astype(jnp.float32)
                  * key_dequant_scale[b0:b0 + BC][..., None]
                  ).reshape(PC * S2, D)
        vf_tab = (value[b0:b0 + BC].astype(jnp.float32)
                  * value_dequant_scale[b0:b0 + BC][..., None]
                  ).reshape(PC * S2, D)
        sidx = sparse_indices[b0:b0 + BC].reshape(PC * NSEL)
        k_sel, v_sel = _build_gather(PC, S2, NSEL, D)(sidx, kf_tab, vf_tab)
        q3 = query[b0:b0 + BC].reshape(PC, GS, D)
        outs.append(_build_attn(PC, GS, L, D)(scale, q3, k_sel, v_sel))
    out = jnp.concatenate(outs, axis=0)
    return out.reshape(B, N1, S1, D)


# speedup vs baseline: 4.6531x; 1.0023x over previous
import functools

import jax
import jax.numpy as jnp
from jax import lax
from jax.experimental import pallas as pl
from jax.experimental.pallas import tpu as pltpu
from jax.experimental.pallas import tpu_sc as plsc

_BLK = 16


@functools.lru_cache(maxsize=None)
def _build_gather(P, S2, NSEL, D):
    TOT = P * NSEL
    L = NSEL * _BLK
    NC, NS = 2, 16
    NW = NC * NS
    assert P == NW
    per_w = NSEL
    TPW = per_w * _BLK
    CHT = 128
    n_chunks = TPW // CHT
    ngrp = per_w // 16
    mesh = plsc.VectorSubcoreMesh(core_axis_name="c", subcore_axis_name="s")

    @functools.partial(
        pl.kernel,
        mesh=mesh,
        out_type=[
            jax.ShapeDtypeStruct((P, L, D), jnp.float32),
            jax.ShapeDtypeStruct((P, L, D), jnp.float32),
        ],
        scratch_types=[
            pltpu.VMEM((per_w,), jnp.int32),
            pltpu.VMEM((TPW,), jnp.int32),
            pltpu.VMEM((CHT, D), jnp.float32),
            pltpu.VMEM((CHT, D), jnp.float32),
            pltpu.VMEM((CHT, D), jnp.float32),
            pltpu.VMEM((CHT, D), jnp.float32),
            pltpu.VMEM((CHT, D), jnp.float32),
            pltpu.VMEM((CHT, D), jnp.float32),
            pltpu.SemaphoreType.DMA,
        ],
    )
    def gather(sidx, kf_tab, vf_tab, k_out, v_out,
               idxv, tix, kb0, vb0, kb1, vb1, kb2, vb2, sem):
        wid = lax.axis_index("c") * NS + lax.axis_index("s")
        base = wid * per_w
        pltpu.sync_copy(sidx.at[pl.ds(base, per_w)], idxv)
        for c in range(ngrp):
            bids = idxv[pl.ds(c * 16, 16)] * _BLK + wid * S2
            for t in range(_BLK):
                tix[pl.ds(t * NSEL + c * 16, 16)] = bids + t
        NB = 3
        kbs, vbs = (kb0, kb1, kb2), (vb0, vb1, vb2)
        copies = [None] * NB
        for g in range(n_chunks + NB - 1):
            if g < n_chunks:
                b = g % NB
                isl = tix.at[pl.ds(g * CHT, CHT)]
                ck = pltpu.async_copy(kf_tab.at[isl], kbs[b], sem)
                cv = pltpu.async_copy(vf_tab.at[isl], vbs[b], sem)
                copies[b] = (ck, cv)
            if g >= NB - 1:
                gp = g - (NB - 1)
                pb = gp % NB
                ckp, cvp = copies[pb]
                ckp.wait()
                cvp.wait()
                toff = gp * CHT
                pltpu.sync_copy(kbs[pb], k_out.at[wid, pl.ds(toff, CHT), :])
                pltpu.sync_copy(vbs[pb], v_out.at[wid, pl.ds(toff, CHT), :])

    return gather


def _attn_body(scale_ref, q_ref, k_ref, v_ref, o_ref):
    q = q_ref[0].astype(jnp.bfloat16)
    kf = k_ref[0].astype(jnp.bfloat16)
    logits = lax.dot_general(q, kf, (((1,), (1,)), ((), ())),
                             preferred_element_type=jnp.float32)
    logits = logits * scale_ref[0]
    m = jnp.max(logits, axis=-1, keepdims=True)
    e = jnp.exp(logits - m)
    den = jnp.sum(e, axis=-1, keepdims=True)
    o = lax.dot_general(e.astype(jnp.bfloat16), v_ref[0].astype(jnp.bfloat16),
                        (((1,), (0,)), ((), ())),
                        preferred_element_type=jnp.float32)
    o_ref[0] = o / den


@functools.lru_cache(maxsize=None)
def _build_attn(P, GS, L, D):
    return pl.pallas_call(
        _attn_body,
        grid=(P,),
        in_specs=[
            pl.BlockSpec(memory_space=pltpu.SMEM),
            pl.BlockSpec((1, GS, D), lambda i: (i, 0, 0)),
            pl.BlockSpec((1, L, D), lambda i: (i, 0, 0)),
            pl.BlockSpec((1, L, D), lambda i: (i, 0, 0)),
        ],
        out_specs=pl.BlockSpec((1, GS, D), lambda i: (i, 0, 0)),
        out_shape=jax.ShapeDtypeStruct((P, GS, D), jnp.float32),
    )


def kernel(query, key, value, sparse_indices, key_dequant_scale,
           value_dequant_scale, scale_value, sparse_block_size):
    B, N1, S1, D = query.shape
    _, N2, S2, _ = key.shape
    G = N1 // N2
    NSEL = sparse_indices.shape[-1]
    P = B * N2
    TOT = P * NSEL
    L = NSEL * _BLK
    GS = G * S1

    scale = jnp.asarray(scale_value, jnp.float32).reshape(1)
    NCH = 2
    BC = B // NCH
    PC = BC * N2
    outs = []
    for c in range(NCH):
        b0 = c * BC
        kf_tab = (key[b0:b0 + BC].astype(jnp.float32)
                  * key_dequant_scale[b0:b0 + BC][..., None]
                  ).reshape(PC * S2, D)
        vf_tab = (value[b0:b0 + BC].astype(jnp.float32)
                  * value_dequant_scale[b0:b0 + BC][..., None]
                  ).reshape(PC * S2, D)
        sidx = sparse_indices[b0:b0 + BC].reshape(PC * NSEL)
        k_sel, v_sel = _build_gather(PC, S2, NSEL, D)(sidx, kf_tab, vf_tab)
        q3 = query[b0:b0 + BC].reshape(PC, GS, D)
        outs.append(_build_attn(PC, GS, L, D)(scale, q3, k_sel, v_sel))
    out = jnp.concatenate(outs, axis=0)
    return out.reshape(B, N1, S1, D)
